# baseline (speedup 1.0000x reference)
import functools

import jax
import jax.numpy as jnp
from jax import lax
from jax.experimental import pallas as pl
from jax.experimental.pallas import tpu as pltpu
from jax.experimental.pallas import tpu_sc as plsc


def kernel(x, table):
    BATCH, HIST = x.shape
    D = table.shape[1]
    G = D // 8
    NT = BATCH // 128

    info = plsc.get_sparse_core_info()
    NW = info.num_cores * info.num_subcores
    RW = BATCH // NW
    TW = NT // NW
    mesh = plsc.VectorSubcoreMesh(core_axis_name="c", subcore_axis_name="s")

    @functools.partial(
        pl.kernel,
        mesh=mesh,
        out_type=jax.ShapeDtypeStruct((HIST, G, NT, 8, 128), jnp.float32),
        scratch_types=[
            pltpu.VMEM((RW, HIST), jnp.int32),
            pltpu.VMEM((2, RW), jnp.int32),
            pltpu.VMEM((2, RW, D), jnp.float32),
            pltpu.VMEM((2, G, TW, 8, 128), jnp.float32),
            pltpu.SemaphoreType.DMA,
            pltpu.SemaphoreType.DMA((2,)),
            pltpu.SemaphoreType.DMA((2,)),
        ],
        compiler_params=pltpu.CompilerParams(
            use_tc_tiling_on_sc=False, needs_layout_passes=False
        ),
    )
    def emb(x_hbm, table_hbm, out_hbm, idx_v, col_v, rows_v, tiles_v,
            sem_i, sem_g, sem_s):
        wid = lax.axis_index("s") * info.num_cores + lax.axis_index("c")
        b0 = wid * RW
        t0 = wid * TW
        lane = lax.iota(jnp.int32, 16)

        pltpu.async_copy(x_hbm.at[pl.ds(b0, RW), :], idx_v, sem_i).wait()

        def build_col(h, b):
            def f(c, carry):
                rows = c * 16 + lane
                vals = plsc.load_gather(
                    idx_v, [rows, jnp.full((16,), h, jnp.int32)]
                )
                col_v[b, pl.ds(c * 16, 16)] = vals
                return carry

            lax.fori_loop(0, RW // 16, f, 0)

        def gather_start(b):
            pltpu.async_copy(table_hbm.at[col_v.at[b]], rows_v.at[b],
                             sem_g.at[b])

        def gather_wait(b):
            pltpu.make_async_copy(
                table_hbm.at[pl.ds(0, RW)], rows_v.at[b], sem_g.at[b]
            ).wait()

        def transpose(b):
            def f(k, carry):
                g = k // (TW * 8)
                tt = (k // 8) % TW
                du = k % 8
                dcol = jnp.full((16,), 8 * g + du, jnp.int32)

                def f2(bc, carry2):
                    rows = tt * 128 + bc * 16 + lane
                    vals = plsc.load_gather(rows_v.at[b], [rows, dcol])
                    tiles_v[b, g, tt, du, pl.ds(bc * 16, 16)] = vals
                    return carry2

                lax.fori_loop(0, 8, f2, 0)
                return carry

            lax.fori_loop(0, G * TW * 8, f, 0)

        def store_start(h, b):
            for g in range(G):
                pltpu.async_copy(
                    tiles_v.at[b, g],
                    out_hbm.at[h, g, pl.ds(t0, TW)],
                    sem_s.at[b],
                )

        def store_wait(b):
            pltpu.make_async_copy(
                tiles_v.at[b], out_hbm.at[0, :, pl.ds(0, TW)], sem_s.at[b]
            ).wait()

        def step(h, b, *, first, last):
            gather_wait(b)
            if not first:
                store_wait(b)
            transpose(b)
            store_start(h, b)
            if not last:
                build_col(h + 2, b)
                gather_start(b)

        build_col(0, 0)
        gather_start(0)
        build_col(1, 1)
        gather_start(1)

        step(0, 0, first=True, last=False)
        step(1, 1, first=True, last=False)

        def body(i, carry):
            step(2 * i + 2, 0, first=False, last=False)
            step(2 * i + 3, 1, first=False, last=False)
            return carry

        lax.fori_loop(0, (HIST - 4) // 2, body, 0)

        step(HIST - 2, 0, first=False, last=True)
        step(HIST - 1, 1, first=False, last=True)
        store_wait(0)
        store_wait(1)

    out5 = emb(x.astype(jnp.int32), table)
    return out5.transpose(2, 4, 0, 1, 3).reshape(BATCH, HIST, D)


# speedup vs baseline: 1.0014x; 1.0014x over previous
import functools

import jax
import jax.numpy as jnp
from jax import lax
from jax.experimental import pallas as pl
from jax.experimental.pallas import tpu as pltpu
from jax.experimental.pallas import tpu_sc as plsc


def kernel(x, table):
    BATCH, HIST = x.shape
    D = table.shape[1]
    G = D // 8
    NT = BATCH // 128

    info = plsc.get_sparse_core_info()
    NW = info.num_cores * info.num_subcores
    RW = BATCH // NW
    TW = NT // NW
    mesh = plsc.VectorSubcoreMesh(core_axis_name="c", subcore_axis_name="s")

    @functools.partial(
        pl.kernel,
        mesh=mesh,
        out_type=jax.ShapeDtypeStruct((HIST, G, NT, 8, 128), jnp.float32),
        scratch_types=[
            pltpu.VMEM((RW, HIST), jnp.int32),
            pltpu.VMEM((2, RW), jnp.int32),
            pltpu.VMEM((2, RW, D), jnp.float32),
            pltpu.VMEM((2, G, TW, 8, 128), jnp.float32),
            pltpu.SemaphoreType.DMA,
            pltpu.SemaphoreType.DMA((2,)),
            pltpu.SemaphoreType.DMA((2,)),
        ],
        compiler_params=pltpu.CompilerParams(
            use_tc_tiling_on_sc=False, needs_layout_passes=False
        ),
    )
    def emb(x_hbm, table_hbm, out_hbm, idx_v, col_v, rows_v, tiles_v,
            sem_i, sem_g, sem_s):
        wid = lax.axis_index("s") * info.num_cores + lax.axis_index("c")
        b0 = wid * RW
        t0 = wid * TW
        lane = lax.iota(jnp.int32, 16)

        pltpu.async_copy(x_hbm.at[pl.ds(b0, RW), :], idx_v, sem_i).wait()

        def build_col(h, b):
            hcol = jnp.full((16,), h, jnp.int32)

            def f(c, carry):
                for s in range(8):
                    rows = c * 128 + (s * 16 + lane)
                    vals = plsc.load_gather(idx_v, [rows, hcol])
                    col_v[b, pl.ds(c * 128 + s * 16, 16)] = vals
                return carry

            lax.fori_loop(0, RW // 128, f, 0)

        def gather_start(b):
            pltpu.async_copy(table_hbm.at[col_v.at[b]], rows_v.at[b],
                             sem_g.at[b])

        def gather_wait(b):
            pltpu.make_async_copy(
                table_hbm.at[pl.ds(0, RW)], rows_v.at[b], sem_g.at[b]
            ).wait()

        def transpose(b):
            def f(g, carry):
                gv = 8 * g
                for du in range(8):
                    dcol = jnp.full((16,), du, jnp.int32) + gv
                    for tt in range(TW):
                        for bc in range(8):
                            rows = tt * 128 + bc * 16 + lane
                            vals = plsc.load_gather(rows_v.at[b], [rows, dcol])
                            tiles_v[b, g, tt, du, pl.ds(bc * 16, 16)] = vals
                return carry

            lax.fori_loop(0, G, f, 0)

        def store_start(h, b):
            for g in range(G):
                pltpu.async_copy(
                    tiles_v.at[b, g],
                    out_hbm.at[h, g, pl.ds(t0, TW)],
                    sem_s.at[b],
                )

        def store_wait(b):
            pltpu.make_async_copy(
                tiles_v.at[b], out_hbm.at[0, :, pl.ds(0, TW)], sem_s.at[b]
            ).wait()

        def step(h, b, *, first, last):
            gather_wait(b)
            if not first:
                store_wait(b)
            transpose(b)
            store_start(h, b)
            if not last:
                build_col(h + 2, b)
                gather_start(b)

        build_col(0, 0)
        gather_start(0)
        build_col(1, 1)
        gather_start(1)

        step(0, 0, first=True, last=False)
        step(1, 1, first=True, last=False)

        def body(i, carry):
            step(2 * i + 2, 0, first=False, last=False)
            step(2 * i + 3, 1, first=False, last=False)
            return carry

        lax.fori_loop(0, (HIST - 4) // 2, body, 0)

        step(HIST - 2, 0, first=False, last=True)
        step(HIST - 1, 1, first=False, last=True)
        store_wait(0)
        store_wait(1)

    out5 = emb(x.astype(jnp.int32), table)
    return out5.transpose(2, 4, 0, 1, 3).reshape(BATCH, HIST, D)


# speedup vs baseline: 1.2211x; 1.2194x over previous
import functools

import jax
import jax.numpy as jnp
from jax import lax
from jax.experimental import pallas as pl
from jax.experimental.pallas import tpu as pltpu
from jax.experimental.pallas import tpu_sc as plsc


def kernel(x, table):
    BATCH, HIST = x.shape
    D = table.shape[1]
    G = D // 8
    NT = BATCH // 128
    SK = D + 1

    info = plsc.get_sparse_core_info()
    NW = info.num_cores * info.num_subcores
    RW = BATCH // NW
    TW = NT // NW
    mesh = plsc.VectorSubcoreMesh(core_axis_name="c", subcore_axis_name="s")

    @functools.partial(
        pl.kernel,
        mesh=mesh,
        out_type=jax.ShapeDtypeStruct((HIST, G, NT, 8, 128), jnp.float32),
        scratch_types=[
            pltpu.VMEM((2, RW), jnp.int32),
            pltpu.VMEM((2, RW, D), jnp.float32),
            pltpu.VMEM((RW * SK,), jnp.float32),
            pltpu.VMEM((2, G, TW, 8, 128), jnp.float32),
            pltpu.SemaphoreType.DMA((2,)),
            pltpu.SemaphoreType.DMA((2,)),
            pltpu.SemaphoreType.DMA((2,)),
        ],
        compiler_params=pltpu.CompilerParams(
            use_tc_tiling_on_sc=False, needs_layout_passes=False
        ),
    )
    def emb(x_hbm, table_hbm, out_hbm, col_v, rows_v, skew_v, tiles_v,
            sem_c, sem_g, sem_s):
        wid = lax.axis_index("s") * info.num_cores + lax.axis_index("c")
        b0 = wid * RW
        t0 = wid * TW
        lane = lax.iota(jnp.int32, 16)

        def col_start(h, b):
            pltpu.async_copy(x_hbm.at[h, pl.ds(b0, RW)], col_v.at[b],
                             sem_c.at[b])

        def col_wait(b):
            pltpu.make_async_copy(x_hbm.at[0, pl.ds(b0, RW)], col_v.at[b],
                                  sem_c.at[b]).wait()

        def gather_start(b):
            pltpu.async_copy(table_hbm.at[col_v.at[b]], rows_v.at[b],
                             sem_g.at[b])

        def gather_wait(b):
            pltpu.make_async_copy(
                table_hbm.at[pl.ds(0, RW)], rows_v.at[b], sem_g.at[b]
            ).wait()

        def transpose(b):
            half0 = lane
            half1 = 16 + lane

            def f1(r, carry):
                base = r * SK
                plsc.store_scatter(skew_v, [base + half0],
                                   rows_v[b, r, pl.ds(0, 16)])
                plsc.store_scatter(skew_v, [base + half1],
                                   rows_v[b, r, pl.ds(16, 16)])
                return carry

            lax.fori_loop(0, RW, f1, 0)

            def f2(g, carry):
                gv = 8 * g
                for du in range(8):
                    dcol = jnp.full((16,), du, jnp.int32) + gv
                    for tt in range(TW):
                        for bc in range(8):
                            rows = (tt * 128 + bc * 16 + lane) * SK
                            vals = plsc.load_gather(skew_v, [rows + dcol])
                            tiles_v[b, g, tt, du, pl.ds(bc * 16, 16)] = vals
                return carry

            lax.fori_loop(0, G, f2, 0)

        def store_start(h, b):
            for g in range(G):
                pltpu.async_copy(
                    tiles_v.at[b, g],
                    out_hbm.at[h, g, pl.ds(t0, TW)],
                    sem_s.at[b],
                )

        def store_wait(b):
            pltpu.make_async_copy(
                tiles_v.at[b], out_hbm.at[0, :, pl.ds(0, TW)], sem_s.at[b]
            ).wait()

        def step(h, b, *, first, last):
            gather_wait(b)
            if not last:
                col_start(h + 2, b)
            if not first:
                store_wait(b)
            transpose(b)
            store_start(h, b)
            if not last:
                col_wait(b)
                gather_start(b)

        col_start(0, 0)
        col_start(1, 1)
        col_wait(0)
        gather_start(0)
        col_wait(1)
        gather_start(1)

        step(0, 0, first=True, last=False)
        step(1, 1, first=True, last=False)

        def body(i, carry):
            step(2 * i + 2, 0, first=False, last=False)
            step(2 * i + 3, 1, first=False, last=False)
            return carry

        lax.fori_loop(0, (HIST - 4) // 2, body, 0)

        step(HIST - 2, 0, first=False, last=True)
        step(HIST - 1, 1, first=False, last=True)
        store_wait(0)
        store_wait(1)

    out5 = emb(x.T.astype(jnp.int32), table)
    return out5.transpose(2, 4, 0, 1, 3).reshape(BATCH, HIST, D)


# speedup vs baseline: 1.2363x; 1.0124x over previous
import functools

import jax
import jax.numpy as jnp
from jax import lax
from jax.experimental import pallas as pl
from jax.experimental.pallas import tpu as pltpu
from jax.experimental.pallas import tpu_sc as plsc


def kernel(x, table):
    BATCH, HIST = x.shape
    D = table.shape[1]
    G = D // 8
    NT = BATCH // 128
    SK = D + 1

    info = plsc.get_sparse_core_info()
    NW = info.num_cores * info.num_subcores
    RW = BATCH // NW
    TW = NT // NW
    mesh = plsc.VectorSubcoreMesh(core_axis_name="c", subcore_axis_name="s")

    @functools.partial(
        pl.kernel,
        mesh=mesh,
        out_type=jax.ShapeDtypeStruct((HIST, G, NT, 8, 128), jnp.float32),
        scratch_types=[
            pltpu.VMEM((2, RW), jnp.int32),
            pltpu.VMEM((2, RW, D), jnp.float32),
            pltpu.VMEM((RW * SK,), jnp.float32),
            pltpu.VMEM((2, G, TW, 8, 128), jnp.float32),
            pltpu.SemaphoreType.DMA((2,)),
            pltpu.SemaphoreType.DMA((2,)),
            pltpu.SemaphoreType.DMA((2,)),
        ],
        compiler_params=pltpu.CompilerParams(
            use_tc_tiling_on_sc=False, needs_layout_passes=False
        ),
    )
    def emb(x_hbm, table_hbm, out_hbm, col_v, rows_v, skew_v, tiles_v,
            sem_c, sem_g, sem_s):
        wid = lax.axis_index("s") * info.num_cores + lax.axis_index("c")
        b0 = wid * RW
        t0 = wid * TW
        lane = lax.iota(jnp.int32, 16)

        def col_start(h, b):
            pltpu.async_copy(x_hbm.at[h, pl.ds(b0, RW)], col_v.at[b],
                             sem_c.at[b])

        def col_wait(b):
            pltpu.make_async_copy(x_hbm.at[0, pl.ds(b0, RW)], col_v.at[b],
                                  sem_c.at[b]).wait()

        def gather_start(b):
            pltpu.async_copy(table_hbm.at[col_v.at[b]], rows_v.at[b],
                             sem_g.at[b])

        def gather_wait(b):
            pltpu.make_async_copy(
                table_hbm.at[pl.ds(0, RW)], rows_v.at[b], sem_g.at[b]
            ).wait()

        def transpose(b):
            half0 = lane
            half1 = 16 + lane

            def f1(r8, carry):
                for j in range(8):
                    r = r8 * 8 + j
                    base = r * SK
                    plsc.store_scatter(skew_v, [base + half0],
                                       rows_v[b, r, pl.ds(0, 16)])
                    plsc.store_scatter(skew_v, [base + half1],
                                       rows_v[b, r, pl.ds(16, 16)])
                return carry

            lax.fori_loop(0, RW // 8, f1, 0)

            def f2(g, carry):
                gv = 8 * g
                for du in range(8):
                    dcol = jnp.full((16,), du, jnp.int32) + gv
                    for tt in range(TW):
                        for bc in range(8):
                            rows = (tt * 128 + bc * 16 + lane) * SK
                            vals = plsc.load_gather(skew_v, [rows + dcol])
                            tiles_v[b, g, tt, du, pl.ds(bc * 16, 16)] = vals
                return carry

            lax.fori_loop(0, G, f2, 0)

        def store_start(h, b):
            for g in range(G):
                pltpu.async_copy(
                    tiles_v.at[b, g],
                    out_hbm.at[h, g, pl.ds(t0, TW)],
                    sem_s.at[b],
                )

        def store_wait(b):
            pltpu.make_async_copy(
                tiles_v.at[b], out_hbm.at[0, :, pl.ds(0, TW)], sem_s.at[b]
            ).wait()

        def step(h, b, *, first, last):
            gather_wait(b)
            if not last:
                col_start(h + 2, b)
            if not first:
                store_wait(b)
            transpose(b)
            store_start(h, b)
            if not last:
                col_wait(b)
                gather_start(b)

        col_start(0, 0)
        col_start(1, 1)
        col_wait(0)
        gather_start(0)
        col_wait(1)
        gather_start(1)

        step(0, 0, first=True, last=False)
        step(1, 1, first=True, last=False)

        def body(i, carry):
            step(2 * i + 2, 0, first=False, last=False)
            step(2 * i + 3, 1, first=False, last=False)
            return carry

        lax.fori_loop(0, (HIST - 4) // 2, body, 0)

        step(HIST - 2, 0, first=False, last=True)
        step(HIST - 1, 1, first=False, last=True)
        store_wait(0)
        store_wait(1)

    out5 = emb(x.T.astype(jnp.int32), table)
    return out5.transpose(2, 4, 0, 1, 3).reshape(BATCH, HIST, D)
